# Initial kernel scaffold; baseline (speedup 1.0000x reference)
import jax
import jax.numpy as jnp
from jax.experimental import pallas as pl


def kernel(pos_src, pos_dst, pos_val, pos_ts, neg_src, neg_dst, neg_val, neg_ts, time_seq, user_embd, item_embd, w_p, w_n, time_table, W1, b1, W2, b2):
    raise NotImplementedError("write your pallas kernel here")


# speedup vs baseline: 2.8144x; 2.8144x over previous
import functools

import jax
import jax.numpy as jnp
from jax import lax
from jax.experimental import pallas as pl
from jax.experimental.pallas import tpu as pltpu
from jax.experimental.pallas import tpu_sc as plsc

_N_USERS = 2000
_N_ITEMS = 3000
_N = 5000
_D = 128
_EP = 160000
_EN = 40000
_MAXT = 1000
_TSTEP = 86400.0
_SLOPE = 0.2

_NC = 2
_NS = 16
_L = 16

_NPAD = 5120
_ROWS_PER_TILE = _NPAD // _NS
_EP_PAD = 163840
_EN_PAD = 40960
_E_PAD = _EP_PAD + _EN_PAD
_TPAD = 1024

_C = 128

_NEG_INF = float("-inf")


def _zero_rows(zbuf, dst_sh, row0, nrows):
  for r in range(16):
    for c in range(8):
      zbuf[r, pl.ds(c * _L, _L)] = jnp.zeros((_L,), jnp.float32)

  def body(i, _):
    pltpu.sync_copy(zbuf, dst_sh.at[pl.ds(row0 + i * 16, 16)])
    return None

  lax.fori_loop(0, nrows // 16, body, None)


_TRASH = _NS * _NPAD


def _init_seg(fillbuf, seg_sh, sid):
  for k in range(1024 // _L):
    fillbuf[pl.ds(k * _L, _L)] = jnp.full((_L,), _NEG_INF, jnp.float32)
  for i in range(_NPAD // 1024):
    pltpu.sync_copy(fillbuf, seg_sh.at[pl.ds(sid * _NPAD + i * 1024, 1024)])


def _segmax_pass(src_h, ts_h, seg_sh, sbuf, tsb, idxg, idx2g, tsg, curg, sem,
                 tab0, base0, nchunks):
  i32 = jnp.int32
  zero16 = jnp.zeros((_L,), i32)
  lseq = lax.iota(i32, _L)
  lanes_gt = [jnp.where(lseq > j, jnp.int32(1), jnp.int32(0))
              for j in range(_L)]

  def chunk_body(ci, _):
    base = base0 + ci * _C
    pltpu.sync_copy(src_h.at[pl.ds(base, _C)], sbuf)
    pltpu.sync_copy(ts_h.at[pl.ds(base, _C)], tsb)
    for k in range(_C // _L):
      s16 = sbuf[pl.ds(k * _L, _L)]
      t16 = tsb[pl.ds(k * _L, _L)]
      gidx = s16 + tab0
      idxg[pl.ds(0, _L)] = gidx
      pltpu.async_copy(seg_sh.at[idxg], curg, sem).wait()
      lose = zero16
      for j in range(_L):
        sj = jnp.broadcast_to(s16[j], (_L,))
        tj = jnp.broadcast_to(t16[j], (_L,))
        bet = jnp.where(tj > t16, jnp.int32(1), jnp.int32(0))
        bet = bet + jnp.where(tj == t16, lanes_gt[j], zero16)
        lose = lose + jnp.where(s16 == sj, bet, zero16)
      pend = jnp.where(t16 > curg[pl.ds(0, _L)], jnp.int32(1), jnp.int32(0))
      live = jnp.where(lose == 0, pend, jnp.int32(0))
      idx2g[pl.ds(0, _L)] = jnp.where(live > 0, gidx, _TRASH)
      tsg[pl.ds(0, _L)] = t16
      pltpu.sync_copy(tsg, seg_sh.at[idx2g])
    return None

  lax.fori_loop(0, nchunks, chunk_body, None)


def _merge_rmax(seg_sh, out_sh, blk_vm, acc_vm, sid):
  plsc.subcore_barrier()
  col0 = sid * _ROWS_PER_TILE
  for k in range(_NS):
    pltpu.sync_copy(seg_sh.at[pl.ds(k * _NPAD + col0, _ROWS_PER_TILE)],
                    blk_vm.at[pl.ds(k * _ROWS_PER_TILE, _ROWS_PER_TILE)])
  for j in range(_ROWS_PER_TILE // _L):
    acc = jnp.full((_L,), _NEG_INF, jnp.float32)
    for k in range(_NS):
      acc = jnp.maximum(acc, blk_vm[pl.ds(k * _ROWS_PER_TILE + j * _L, _L)])
    acc = jnp.where(acc > _NEG_INF, acc, 0.0)
    acc_vm[pl.ds(j * _L, _L)] = acc
  pltpu.sync_copy(acc_vm, out_sh.at[pl.ds(col0, _ROWS_PER_TILE)])
  plsc.subcore_barrier()


def _edge_weight_loop(src_h, dst_h, ts_h, val_h, tq_h, w_h, tt_h, we_out,
                      rmax_sh, tagg_sh, bufs, sem, base0, nchunks, sign):
  sbuf, dbuf, tsbuf, vbuf, qbuf, linbuf, wbuf, webuf, rmbuf, ttrows = bufs

  def chunk_body(ci, _):
    base = base0 + ci * _C
    pltpu.sync_copy(src_h.at[pl.ds(base, _C)], sbuf)
    pltpu.sync_copy(dst_h.at[pl.ds(base, _C)], dbuf)
    pltpu.sync_copy(ts_h.at[pl.ds(base, _C)], tsbuf)
    pltpu.sync_copy(val_h.at[pl.ds(base, _C)], vbuf)
    pltpu.sync_copy(tq_h.at[pl.ds(base, _C)], qbuf)
    for k in range(_C // _L):
      s16 = sbuf[pl.ds(k * _L, _L)]
      d16 = dbuf[pl.ds(k * _L, _L)]
      linbuf[pl.ds(k * _L, _L)] = s16 * _N + d16
    pltpu.async_copy(w_h.at[linbuf], wbuf, sem).wait()
    pltpu.async_copy(rmax_sh.at[sbuf], rmbuf, sem).wait()
    for k in range(_C // _L):
      rm = rmbuf[pl.ds(k * _L, _L)]
      t = jnp.exp((tsbuf[pl.ds(k * _L, _L)] - rm) / _TSTEP)
      webuf[pl.ds(k * _L, _L)] = vbuf[pl.ds(k * _L, _L)] * t * wbuf[pl.ds(k * _L, _L)] * sign
    pltpu.sync_copy(webuf, we_out.at[pl.ds(base, _C)])
    pltpu.async_copy(tt_h.at[qbuf], ttrows, sem).wait()

    def scale_group(g, _):
      wvec = webuf[pl.ds(g * _L, _L)]
      for lane in range(_L):
        e = g * _L + lane
        wb = jnp.broadcast_to(wvec[lane], (_L,))
        for c in range(_D // _L):
          ttrows[e, pl.ds(c * _L, _L)] = ttrows[e, pl.ds(c * _L, _L)] * wb
      return None

    lax.fori_loop(0, _C // _L, scale_group, None)
    pltpu.sync_copy(ttrows, tagg_sh.at[dbuf], add=True)
    return None

  lax.fori_loop(0, nchunks, chunk_body, None)


def _edge_kernel(src_h, dst_h, ts_h, val_h, tq_h, wp_h, wn_h, tt_h,
                 we_out, tagg_out,
                 blk_vm, acc_vm, fillbuf,
                 sbuf, dbuf, tsbuf, vbuf, qbuf, linbuf, wbuf, webuf, rmbuf,
                 idxg, idx2g, tsg, curg, ttrows, zbuf,
                 seg_sh, rmaxp_sh, rmaxn_sh, tagg_sh, sem):
  cid = lax.axis_index("c")
  sid = lax.axis_index("s")
  tab0 = sid * _NPAD

  _init_seg(fillbuf, seg_sh, sid)
  plsc.subcore_barrier()
  _segmax_pass(src_h, ts_h, seg_sh, sbuf, tsbuf, idxg, idx2g, tsg, curg, sem,
               tab0, sid * (_EP_PAD // _NS), (_EP_PAD // _NS) // _C)
  _merge_rmax(seg_sh, rmaxp_sh, blk_vm, acc_vm, sid)

  _init_seg(fillbuf, seg_sh, sid)
  plsc.subcore_barrier()
  _segmax_pass(src_h, ts_h, seg_sh, sbuf, tsbuf, idxg, idx2g, tsg, curg, sem,
               tab0, _EP_PAD + sid * (_EN_PAD // _NS), (_EN_PAD // _NS) // _C)
  _merge_rmax(seg_sh, rmaxn_sh, blk_vm, acc_vm, sid)

  _zero_rows(zbuf, tagg_sh, sid * _ROWS_PER_TILE, _ROWS_PER_TILE)
  plsc.subcore_barrier()

  bufs = (sbuf, dbuf, tsbuf, vbuf, qbuf, linbuf, wbuf, webuf, rmbuf, ttrows)
  pos_half = _EP_PAD // _NC
  pos_tile = pos_half // _NS
  _edge_weight_loop(src_h, dst_h, ts_h, val_h, tq_h, wp_h, tt_h, we_out,
                    rmaxp_sh, tagg_sh, bufs, sem,
                    cid * pos_half + sid * pos_tile, pos_tile // _C, 1.0)
  neg_half = _EN_PAD // _NC
  neg_tile = neg_half // _NS
  _edge_weight_loop(src_h, dst_h, ts_h, val_h, tq_h, wn_h, tt_h, we_out,
                    rmaxn_sh, tagg_sh, bufs, sem,
                    _EP_PAD + cid * neg_half + sid * neg_tile,
                    neg_tile // _C, -1.0)
  plsc.subcore_barrier()
  pltpu.sync_copy(tagg_sh.at[pl.ds(sid * _ROWS_PER_TILE, _ROWS_PER_TILE)],
                  tagg_out.at[cid, pl.ds(sid * _ROWS_PER_TILE, _ROWS_PER_TILE)])


def _layer_kernel(h_h, src_h, dst_h, we_h, agg_out,
                  sbuf, dbuf, webuf, rows, zbuf, h_sh, agg_sh, sem):
  cid = lax.axis_index("c")
  sid = lax.axis_index("s")
  row0 = sid * _ROWS_PER_TILE
  pltpu.sync_copy(h_h.at[pl.ds(row0, _ROWS_PER_TILE)],
                  h_sh.at[pl.ds(row0, _ROWS_PER_TILE)])
  _zero_rows(zbuf, agg_sh, row0, _ROWS_PER_TILE)
  plsc.subcore_barrier()

  half = _E_PAD // _NC
  per_tile = half // _NS
  base0 = cid * half + sid * per_tile

  def chunk_body(ci, _):
    base = base0 + ci * _C
    pltpu.sync_copy(src_h.at[pl.ds(base, _C)], sbuf)
    pltpu.sync_copy(dst_h.at[pl.ds(base, _C)], dbuf)
    pltpu.sync_copy(we_h.at[pl.ds(base, _C)], webuf)
    pltpu.async_copy(h_sh.at[sbuf], rows, sem).wait()

    def scale_group(g, _):
      wvec = webuf[pl.ds(g * _L, _L)]
      for lane in range(_L):
        e = g * _L + lane
        wb = jnp.broadcast_to(wvec[lane], (_L,))
        for c in range(_D // _L):
          rows[e, pl.ds(c * _L, _L)] = rows[e, pl.ds(c * _L, _L)] * wb
      return None

    lax.fori_loop(0, _C // _L, scale_group, None)
    pltpu.sync_copy(rows, agg_sh.at[dbuf], add=True)
    return None

  lax.fori_loop(0, per_tile // _C, chunk_body, None)
  plsc.subcore_barrier()
  pltpu.sync_copy(agg_sh.at[pl.ds(row0, _ROWS_PER_TILE)],
                  agg_out.at[cid, pl.ds(row0, _ROWS_PER_TILE)])


def _dense_body(t0, t1, a0, a1, w, b, h_out, hn_out):
  x = t0[...] + t1[...] + a0[...] + a1[...]
  y = jnp.dot(x, w[...], preferred_element_type=jnp.float32,
              precision=lax.Precision.HIGHEST) + b[...]
  h = jnp.where(y >= 0.0, y, _SLOPE * y)
  nrm = jnp.sqrt(jnp.sum(h * h, axis=1, keepdims=True))
  h_out[...] = h
  hn_out[...] = h / jnp.maximum(nrm, 1e-12)


_SC_MESH = plsc.VectorSubcoreMesh(core_axis_name="c", subcore_axis_name="s",
                                  num_cores=_NC, num_subcores=_NS)

_edge_call = pl.kernel(
    _edge_kernel,
    out_type=(
        jax.ShapeDtypeStruct((_E_PAD,), jnp.float32),
        jax.ShapeDtypeStruct((_NC, _NPAD, _D), jnp.float32),
    ),
    mesh=_SC_MESH,
    scratch_types=[
        pltpu.VMEM((_NS * _ROWS_PER_TILE,), jnp.float32),
        pltpu.VMEM((_ROWS_PER_TILE,), jnp.float32),
        pltpu.VMEM((1024,), jnp.float32),
        pltpu.VMEM((_C,), jnp.int32),
        pltpu.VMEM((_C,), jnp.int32),
        pltpu.VMEM((_C,), jnp.float32),
        pltpu.VMEM((_C,), jnp.float32),
        pltpu.VMEM((_C,), jnp.int32),
        pltpu.VMEM((_C,), jnp.int32),
        pltpu.VMEM((_C,), jnp.float32),
        pltpu.VMEM((_C,), jnp.float32),
        pltpu.VMEM((_C,), jnp.float32),
        pltpu.VMEM((_L,), jnp.int32),
        pltpu.VMEM((_L,), jnp.int32),
        pltpu.VMEM((_L,), jnp.float32),
        pltpu.VMEM((_L,), jnp.float32),
        pltpu.VMEM((_C, _D), jnp.float32),
        pltpu.VMEM((16, _D), jnp.float32),
        pltpu.VMEM_SHARED((_NS * _NPAD + _C,), jnp.float32),
        pltpu.VMEM_SHARED((_NPAD,), jnp.float32),
        pltpu.VMEM_SHARED((_NPAD,), jnp.float32),
        pltpu.VMEM_SHARED((_NPAD, _D), jnp.float32),
        pltpu.SemaphoreType.DMA,
    ],
)

_layer_call = pl.kernel(
    _layer_kernel,
    out_type=jax.ShapeDtypeStruct((_NC, _NPAD, _D), jnp.float32),
    mesh=_SC_MESH,
    scratch_types=[
        pltpu.VMEM((_C,), jnp.int32),
        pltpu.VMEM((_C,), jnp.int32),
        pltpu.VMEM((_C,), jnp.float32),
        pltpu.VMEM((_C, _D), jnp.float32),
        pltpu.VMEM((16, _D), jnp.float32),
        pltpu.VMEM_SHARED((_NPAD, _D), jnp.float32),
        pltpu.VMEM_SHARED((_NPAD, _D), jnp.float32),
        pltpu.SemaphoreType.DMA,
    ],
)

_dense_call = pl.pallas_call(
    _dense_body,
    grid=(_NPAD // 640,),
    in_specs=[
        pl.BlockSpec((640, _D), lambda i: (i, 0)),
        pl.BlockSpec((640, _D), lambda i: (i, 0)),
        pl.BlockSpec((640, _D), lambda i: (i, 0)),
        pl.BlockSpec((640, _D), lambda i: (i, 0)),
        pl.BlockSpec((_D, _D), lambda i: (0, 0)),
        pl.BlockSpec((1, _D), lambda i: (0, 0)),
    ],
    out_specs=[
        pl.BlockSpec((640, _D), lambda i: (i, 0)),
        pl.BlockSpec((640, _D), lambda i: (i, 0)),
    ],
    out_shape=[
        jax.ShapeDtypeStruct((_NPAD, _D), jnp.float32),
        jax.ShapeDtypeStruct((_NPAD, _D), jnp.float32),
    ],
)


def _pad1(x, total, fill):
  return jnp.concatenate(
      [x, jnp.full((total - x.shape[0],), fill, x.dtype)], axis=0)


def kernel(pos_src, pos_dst, pos_val, pos_ts, neg_src, neg_dst, neg_val,
           neg_ts, time_seq, user_embd, item_embd, w_p, w_n, time_table,
           W1, b1, W2, b2):
  i32 = jnp.int32
  src_all = jnp.concatenate([
      _pad1(pos_src.astype(i32), _EP_PAD, 0),
      _pad1(neg_src.astype(i32), _EN_PAD, 0)])
  dst_all = jnp.concatenate([
      _pad1(pos_dst.astype(i32), _EP_PAD, 0),
      _pad1(neg_dst.astype(i32), _EN_PAD, 0)])
  ts_all = jnp.concatenate([
      _pad1(pos_ts, _EP_PAD, _NEG_INF),
      _pad1(neg_ts, _EN_PAD, _NEG_INF)])
  val_all = jnp.concatenate([
      _pad1(pos_val, _EP_PAD, 0.0),
      _pad1(neg_val, _EN_PAD, 0.0)])
  tq_all = jnp.concatenate([
      _pad1(time_seq[:_EP].astype(i32), _EP_PAD, 0),
      _pad1(time_seq[_EP:].astype(i32), _EN_PAD, 0)])
  tt_pad = jnp.concatenate(
      [time_table, jnp.zeros((_TPAD - _MAXT, _D), jnp.float32)], axis=0)

  we, tagg = _edge_call(src_all, dst_all, ts_all, val_all, tq_all,
                        w_p.reshape(-1), w_n.reshape(-1), tt_pad)

  h0 = jnp.concatenate([user_embd, item_embd], axis=0)
  h = jnp.concatenate([h0, jnp.zeros((_NPAD - _N, _D), jnp.float32)], axis=0)

  hns = []
  for (w, b) in ((W1, b1), (W2, b2)):
    part = _layer_call(h, src_all, dst_all, we)
    h, hn = _dense_call(tagg[0], tagg[1], part[0], part[1], w,
                        b.reshape(1, _D))
    hns.append(hn[:_N])

  return jnp.concatenate([h0, hns[0], hns[1]], axis=1)
